# baseline (speedup 1.0000x reference)
import functools
import numpy as np
import jax
import jax.numpy as jnp
from jax import lax
from jax.experimental import pallas as pl
from jax.experimental.pallas import tpu as pltpu
from jax.experimental.pallas import tpu_sc as plsc

N_ELEC = 10000
N_NUC = 16
EMB = 128
KER = 64
DFD = 32
CUTOFF = 10.0
N_LAYERS = 3
LOG2 = float(np.log(2.0))

_delta = 1.0 / (2 * DFD)
_qs = np.linspace(_delta, 1.0 - _delta, DFD)
_mus = CUTOFF * _qs ** 2
_sigmas = (1.0 + CUTOFF * _qs) / 7.0
_MUS = np.zeros((1, 128), np.float32)
_MUS[0, :DFD] = _mus
_ISIG2 = np.zeros((1, 128), np.float32)
_ISIG2[0, :DFD] = 1.0 / _sigmas ** 2
_BMASK = np.zeros((1, 128), np.float32)
_BMASK[0, :DFD] = 1.0

EDGE_B = 2048
NODE_B = 2000

NTILES = 32
NBATCH = 84
BATCH = 64
EP = NTILES * NBATCH * BATCH
NACC = 10240
ZROWS = NACC // 16


def _ssp(x):
    return jnp.logaddexp(x, 0.0) - LOG2


def _pad2(w, r, c):
    return jnp.zeros((r, c), jnp.float32).at[: w.shape[0], : w.shape[1]].set(w)


def _pad_row(b, c):
    return jnp.zeros((1, c), jnp.float32).at[0, : b.shape[0]].set(b)


def _bdot(a, b_ref):
    return jnp.dot(a.astype(jnp.bfloat16), b_ref[:].astype(jnp.bfloat16),
                   preferred_element_type=jnp.float32)


def _edge_body(d_ref, mus_ref, isig_ref, bmask_ref,
               w1_ref, b1_ref, w2_ref, b2_ref, w3_ref, out_ref):
    d = d_ref[:]
    env = d * d * jnp.exp(-d)
    t = d - mus_ref[:]
    feat = env * jnp.exp(-(t * t) * isig_ref[:]) * bmask_ref[:]
    h1 = _ssp(_bdot(feat, w1_ref) + b1_ref[:])
    h2 = _ssp(_bdot(h1, w2_ref) + b2_ref[:])
    out_ref[:] = _bdot(h2, w3_ref)


def _edge_ne_body(d_ref, s_ref, mus_ref, isig_ref, bmask_ref,
                  w1_ref, b1_ref, w2_ref, b2_ref, w3_ref, y_ref, out_ref):
    d = d_ref[:]
    env = d * d * jnp.exp(-d)
    t = d - mus_ref[:]
    feat = env * jnp.exp(-(t * t) * isig_ref[:]) * bmask_ref[:]
    h1 = _ssp(_bdot(feat, w1_ref) + b1_ref[:])
    h2 = _ssp(_bdot(h1, w2_ref) + b2_ref[:])
    we = _bdot(h2, w3_ref)
    s = s_ref[:]
    lanes = lax.broadcasted_iota(jnp.int32, (s.shape[0], 128), 1)
    onehot = (lanes == s).astype(jnp.float32)
    hx = jnp.dot(onehot, y_ref[:], preferred_element_type=jnp.float32)
    out_ref[:] = we * hx


def _node_h_body(e_ref, w1s_ref, b1s_ref, w2s_ref,
                 w1a_ref, b1a_ref, w2a_ref, hs_ref, ha_ref):
    e = e_ref[:]
    hs = _ssp(jnp.dot(e, w1s_ref[:], preferred_element_type=jnp.float32) + b1s_ref[:])
    hs_ref[:] = jnp.dot(hs, w2s_ref[:], preferred_element_type=jnp.float32)
    ha = _ssp(jnp.dot(e, w1a_ref[:], preferred_element_type=jnp.float32) + b1a_ref[:])
    ha_ref[:] = jnp.dot(ha, w2a_ref[:], preferred_element_type=jnp.float32)


def _update_body(e_ref, zs_ref, za_ref, zn_ref,
                 ws1_ref, bs1_ref, ws2_ref,
                 wa1_ref, ba1_ref, wa2_ref,
                 wn1_ref, bn1_ref, wn2_ref, out_ref):
    acc = e_ref[:]
    for z_ref, w1_ref, b1_ref, w2_ref in (
            (zs_ref, ws1_ref, bs1_ref, ws2_ref),
            (za_ref, wa1_ref, ba1_ref, wa2_ref),
            (zn_ref, wn1_ref, bn1_ref, wn2_ref)):
        z = z_ref[0] + z_ref[1]
        h = _ssp(jnp.dot(z, w1_ref[:], preferred_element_type=jnp.float32) + b1_ref[:])
        acc = acc + jnp.dot(h, w2_ref[:], preferred_element_type=jnp.float32)
    out_ref[:] = acc


def _full(shape):
    return pl.BlockSpec(shape, lambda i: tuple(0 for _ in shape))


def _edge_mlp(dists2d, wts, e_total):
    grid = (e_total // EDGE_B,)
    in_specs = [pl.BlockSpec((EDGE_B, 1), lambda i: (i, 0)),
                _full((1, 128)), _full((1, 128)), _full((1, 128)),
                _full((128, 128)), _full((1, 128)),
                _full((128, 128)), _full((1, 128)),
                _full((128, 128))]
    return pl.pallas_call(
        _edge_body, grid=grid, in_specs=in_specs,
        out_specs=pl.BlockSpec((EDGE_B, 128), lambda i: (i, 0)),
        out_shape=jax.ShapeDtypeStruct((e_total, 128), jnp.float32),
    )(dists2d, jnp.asarray(_MUS), jnp.asarray(_ISIG2), jnp.asarray(_BMASK), *wts)


def _edge_mlp_ne(dists2d, senders2d, wts, y128, e_total):
    grid = (e_total // EDGE_B,)
    in_specs = [pl.BlockSpec((EDGE_B, 1), lambda i: (i, 0)),
                pl.BlockSpec((EDGE_B, 1), lambda i: (i, 0)),
                _full((1, 128)), _full((1, 128)), _full((1, 128)),
                _full((128, 128)), _full((1, 128)),
                _full((128, 128)), _full((1, 128)),
                _full((128, 128)), _full((128, 128))]
    return pl.pallas_call(
        _edge_ne_body, grid=grid, in_specs=in_specs,
        out_specs=pl.BlockSpec((EDGE_B, 128), lambda i: (i, 0)),
        out_shape=jax.ShapeDtypeStruct((e_total, 128), jnp.float32),
    )(dists2d, senders2d, jnp.asarray(_MUS), jnp.asarray(_ISIG2),
      jnp.asarray(_BMASK), *wts, y128)


def _node_h(elec, wts):
    grid = (N_ELEC // NODE_B,)
    in_specs = [pl.BlockSpec((NODE_B, 128), lambda i: (i, 0)),
                _full((128, 128)), _full((1, 128)), _full((128, 128)),
                _full((128, 128)), _full((1, 128)), _full((128, 128))]
    out_specs = [pl.BlockSpec((NODE_B, 128), lambda i: (i, 0))] * 2
    return pl.pallas_call(
        _node_h_body, grid=grid, in_specs=in_specs, out_specs=out_specs,
        out_shape=[jax.ShapeDtypeStruct((N_ELEC, 128), jnp.float32)] * 2,
    )(elec, *wts)


def _update(elec, zs, za, zn, wts):
    grid = (N_ELEC // NODE_B,)
    in_specs = [pl.BlockSpec((NODE_B, 128), lambda i: (i, 0))] + \
               [pl.BlockSpec((2, NODE_B, 128), lambda i: (0, i, 0))] * 3 + \
               [_full((128, 128)), _full((1, 128)), _full((128, 128))] * 3
    return pl.pallas_call(
        _update_body, grid=grid, in_specs=in_specs,
        out_specs=pl.BlockSpec((NODE_B, 128), lambda i: (i, 0)),
        out_shape=jax.ShapeDtypeStruct((N_ELEC, EMB), jnp.float32),
    )(elec, zs, za, zn, *wts)


def _sc_body(gather_flags):
  def outer_body(ws, wa, wn, h_s, h_a, ss3, sa3, rs3, ra3, rn3,
                 o_s, o_a, o_n, acc, ridx, sidx, bufw0, bufw1, bufh):
    @functools.partial(pl.run_scoped,
                       semw0=pltpu.SemaphoreType.DMA(()),
                       semw1=pltpu.SemaphoreType.DMA(()),
                       semh=pltpu.SemaphoreType.DMA(()))
    def body(semw0, semw1, semh):
      _sc_impl(gather_flags, ws, wa, wn, h_s, h_a, ss3, sa3, rs3, ra3, rn3,
               o_s, o_a, o_n, acc, ridx, sidx,
               bufw0, bufw1, bufh, semw0, semw1, semh)
  return outer_body


def _sc_impl(gather_flags, ws, wa, wn, h_s, h_a, ss3, sa3, rs3, ra3, rn3,
             o_s, o_a, o_n, acc, ridx, sidx,
             bufw0, bufw1, bufh, semw0, semw1, semh):
    c = lax.axis_index("c")
    s = lax.axis_index("s")
    w = s * 2 + c
    bufws = (bufw0, bufw1)
    semws = (semw0, semw1)
    xbufs = (bufw0, bufw1, bufh)
    xsems = (semw0, semw1, semh)

    def zero_buf_row(i, carry):
        for cc in range(8):
            bufw0[i, pl.ds(cc * 16, 16)] = jnp.zeros((16,), jnp.float32)
        return carry

    def _zero():
        for k in range(ZROWS // BATCH):
            pltpu.sync_copy(bufw0, acc.at[pl.ds(s * ZROWS + k * BATCH, BATCH)])

    def _writeout(o):
        pltpu.sync_copy(acc.at[pl.ds(s * ZROWS, ZROWS)],
                        o.at[c, pl.ds(s * ZROWS, ZROWS)])

    def _scatter(weh, r3, do_gather, h, s3):
        pltpu.sync_copy(r3.at[w], ridx)
        if do_gather:
            pltpu.sync_copy(s3.at[w], sidx)
        base0 = w * NBATCH

        def load_desc(j, b):
            return pltpu.make_async_copy(
                weh.at[pl.ds((base0 + j) * BATCH, BATCH)], xbufs[b], xsems[b])

        def gath_desc(j, b):
            return pltpu.make_async_copy(h.at[sidx.at[j]], xbufs[b], xsems[b])

        if not do_gather:
            def do_batch(j, k):
                load_desc(j, k).wait()
                pltpu.sync_copy(bufws[k], acc.at[ridx.at[j]], add=True)

                @pl.when(j + 2 < NBATCH)
                def _():
                    load_desc(j + 2, k).start()

            load_desc(0, 0).start()
            load_desc(1, 1).start()

            def loop_body(tt, carry):
                do_batch(2 * tt, 0)
                do_batch(2 * tt + 1, 1)
                return carry
            lax.fori_loop(0, NBATCH // 2, loop_body, 0)
        else:
            def do_step(j, bi):
                bh = (bi + 1) % 3
                bn = (bi + 2) % 3
                gath_desc(j, bh).wait()

                @pl.when(j + 1 < NBATCH)
                def _():
                    gath_desc(j + 1, bn).start()
                load_desc(j, bi).wait()

                def mul_row(i, carry2):
                    for cc in range(4):
                        sl = pl.ds(cc * 16, 16)
                        xbufs[bi][i, sl] = xbufs[bi][i, sl] * xbufs[bh][i, sl]
                    return carry2
                lax.fori_loop(0, BATCH, mul_row, 0)

                @pl.when(j + 1 < NBATCH)
                def _():
                    load_desc(j + 1, bh).start()
                pltpu.sync_copy(xbufs[bi], acc.at[ridx.at[j]], add=True)

            load_desc(0, 0).start()
            gath_desc(0, 1).start()

            def loop_body(tt, carry):
                do_step(3 * tt, 0)
                do_step(3 * tt + 1, 1)
                do_step(3 * tt + 2, 2)
                return carry
            lax.fori_loop(0, NBATCH // 3, loop_body, 0)

    for weh, r3, o, do_gather, h, s3 in (
            (ws, rs3, o_s, gather_flags[0], h_s, ss3),
            (wa, ra3, o_a, gather_flags[1], h_a, sa3),
            (wn, rn3, o_n, gather_flags[2], None, None)):
        lax.fori_loop(0, BATCH, zero_buf_row, 0)
        _zero()
        plsc.subcore_barrier()
        _scatter(weh, r3, do_gather, h, s3)
        plsc.subcore_barrier()
        _writeout(o)
        plsc.subcore_barrier()


def _make_sc(gather_flags, interpret=False):
  return functools.partial(
      pl.kernel,
      mesh=plsc.VectorSubcoreMesh(core_axis_name="c", subcore_axis_name="s"),
      out_type=[jax.ShapeDtypeStruct((2, NACC, 128), jnp.float32)] * 3,
      scratch_types=[pltpu.VMEM_SHARED((NACC, 128), jnp.float32),
                     pltpu.VMEM((NBATCH, BATCH), jnp.int32),
                     pltpu.VMEM((NBATCH, BATCH), jnp.int32),
                     pltpu.VMEM((BATCH, 128), jnp.float32),
                     pltpu.VMEM((BATCH, 128), jnp.float32),
                     pltpu.VMEM((BATCH, 128), jnp.float32)],
      interpret=interpret,
  )(_sc_body(gather_flags))


_seg3 = _make_sc((False, False, False))
_seg3g = _make_sc((True, True, False))


def _prep_w_mlp(layers, fold_row=None):
    w1 = _pad2(layers[0]['W'], 128, 128)
    b1 = _pad_row(layers[0]['b'], 128)
    w2 = _pad2(layers[1]['W'], 128, 128)
    b2 = _pad_row(layers[1]['b'], 128)
    w3 = layers[2]['W']
    if fold_row is not None:
        w3 = w3 * fold_row[None, :]
    w3 = _pad2(w3, 128, 128)
    return (w1, b1, w2, b2, w3)


def _prep_h_mlp(layers):
    return (_pad2(layers[0]['W'], 128, 128), _pad_row(layers[0]['b'], 128),
            _pad2(layers[1]['W'], 128, 128))


def _prep_g_mlp(layers):
    return (_pad2(layers[0]['W'], 128, 128), _pad_row(layers[0]['b'], 128),
            _pad2(layers[1]['W'], 128, 128))


def kernel(dists_same, dists_anti, dists_ne, senders_same, receivers_same,
           senders_anti, receivers_anti, senders_ne, receivers_ne, params):
    E = dists_same.shape[0]
    pad = EP - E

    def _pedge(x, val=0):
        return jnp.pad(x, (0, pad), constant_values=val)

    ds2 = _pedge(dists_same).reshape(EP, 1)
    da2 = _pedge(dists_anti).reshape(EP, 1)
    dn2 = _pedge(dists_ne).reshape(EP, 1)
    sn2 = _pedge(senders_ne).reshape(EP, 1)
    ss3 = _pedge(senders_same).reshape(NTILES, NBATCH, BATCH)
    sa3 = _pedge(senders_anti).reshape(NTILES, NBATCH, BATCH)
    rs3 = _pedge(receivers_same, N_ELEC).reshape(NTILES, NBATCH, BATCH)
    ra3 = _pedge(receivers_anti, N_ELEC).reshape(NTILES, NBATCH, BATCH)
    rn3 = _pedge(receivers_ne, N_ELEC).reshape(NTILES, NBATCH, BATCH)
    y128 = _pad2(params['Y'], 128, 128)

    elec = jnp.broadcast_to(params['X'][0], (N_ELEC, EMB))

    lps = params['layers']
    we = []
    for i in range(N_LAYERS):
        lp = lps[i]
        fold_s = lp['h_same'][0] if i == 0 else None
        fold_a = lp['h_anti'][0] if i == 0 else None
        we.append((_edge_mlp(ds2, _prep_w_mlp(lp['w_same'], fold_s), EP),
                   _edge_mlp(da2, _prep_w_mlp(lp['w_anti'], fold_a), EP),
                   _edge_mlp_ne(dn2, sn2, _prep_w_mlp(lp['w_ne']), y128, EP)))

    for i in range(N_LAYERS):
        lp = lps[i]
        we_s, we_a, weh_n = we[i]
        if i == 0:
            z_s, z_a, z_n = _seg3(we_s, we_a, weh_n, elec, elec,
                                  ss3, sa3, rs3, ra3, rn3)
        else:
            h_s, h_a = _node_h(elec, _prep_h_mlp(lp['h_same']) + _prep_h_mlp(lp['h_anti']))
            z_s, z_a, z_n = _seg3g(we_s, we_a, weh_n, h_s, h_a,
                                   ss3, sa3, rs3, ra3, rn3)
        elec = _update(elec, z_s, z_a, z_n,
                       _prep_g_mlp(lp['g_same']) + _prep_g_mlp(lp['g_anti'])
                       + _prep_g_mlp(lp['g_ne']))
    return elec


# speedup vs baseline: 1.4103x; 1.4103x over previous
import functools
import numpy as np
import jax
import jax.numpy as jnp
from jax import lax
from jax.experimental import pallas as pl
from jax.experimental.pallas import tpu as pltpu
from jax.experimental.pallas import tpu_sc as plsc

N_ELEC = 10000
N_NUC = 16
EMB = 128
KER = 64
DFD = 32
CUTOFF = 10.0
N_LAYERS = 3
LOG2 = float(np.log(2.0))

_delta = 1.0 / (2 * DFD)
_qs = np.linspace(_delta, 1.0 - _delta, DFD)
_mus = CUTOFF * _qs ** 2
_sigmas = (1.0 + CUTOFF * _qs) / 7.0
_MUS = np.zeros((1, 128), np.float32)
_MUS[0, :DFD] = _mus
_ISIG2 = np.zeros((1, 128), np.float32)
_ISIG2[0, :DFD] = 1.0 / _sigmas ** 2
_BMASK = np.zeros((1, 128), np.float32)
_BMASK[0, :DFD] = 1.0

EDGE_B = 2048
NODE_B = 2000

NTILES = 32
NBATCH = 80
BATCH = 64
EP = NTILES * NBATCH * BATCH
NACC = 10240
ZROWS = NACC // 16


def _ssp(x):
    return jnp.logaddexp(x, 0.0) - LOG2


def _pad2(w, r, c):
    return jnp.zeros((r, c), jnp.float32).at[: w.shape[0], : w.shape[1]].set(w)


def _pad_row(b, c):
    return jnp.zeros((1, c), jnp.float32).at[0, : b.shape[0]].set(b)


def _bdot(a, b_ref):
    return jnp.dot(a.astype(jnp.bfloat16), b_ref[:].astype(jnp.bfloat16),
                   preferred_element_type=jnp.float32)


def _edge_body(d_ref, mus_ref, isig_ref, bmask_ref,
               w1_ref, b1_ref, w2_ref, b2_ref, w3_ref, out_ref):
    d = d_ref[:]
    env = d * d * jnp.exp(-d)
    t = d - mus_ref[:]
    feat = env * jnp.exp(-(t * t) * isig_ref[:]) * bmask_ref[:]
    h1 = _ssp(_bdot(feat, w1_ref) + b1_ref[:])
    h2 = _ssp(_bdot(h1, w2_ref) + b2_ref[:])
    out_ref[:] = _bdot(h2, w3_ref)


def _edge_ne_body(d_ref, s_ref, mus_ref, isig_ref, bmask_ref,
                  w1_ref, b1_ref, w2_ref, b2_ref, w3_ref, y_ref, out_ref):
    d = d_ref[:]
    env = d * d * jnp.exp(-d)
    t = d - mus_ref[:]
    feat = env * jnp.exp(-(t * t) * isig_ref[:]) * bmask_ref[:]
    h1 = _ssp(_bdot(feat, w1_ref) + b1_ref[:])
    h2 = _ssp(_bdot(h1, w2_ref) + b2_ref[:])
    we = _bdot(h2, w3_ref)
    s = s_ref[:]
    lanes = lax.broadcasted_iota(jnp.int32, (s.shape[0], 128), 1)
    onehot = (lanes == s).astype(jnp.float32)
    hx = jnp.dot(onehot, y_ref[:], preferred_element_type=jnp.float32)
    out_ref[:] = we * hx


def _node_h_body(e_ref, w1s_ref, b1s_ref, w2s_ref,
                 w1a_ref, b1a_ref, w2a_ref, hs_ref, ha_ref):
    e = e_ref[:]
    hs = _ssp(jnp.dot(e, w1s_ref[:], preferred_element_type=jnp.float32) + b1s_ref[:])
    hs_ref[:] = jnp.dot(hs, w2s_ref[:], preferred_element_type=jnp.float32)
    ha = _ssp(jnp.dot(e, w1a_ref[:], preferred_element_type=jnp.float32) + b1a_ref[:])
    ha_ref[:] = jnp.dot(ha, w2a_ref[:], preferred_element_type=jnp.float32)


def _update_body(e_ref, zs_ref, za_ref, zn_ref,
                 ws1_ref, bs1_ref, ws2_ref,
                 wa1_ref, ba1_ref, wa2_ref,
                 wn1_ref, bn1_ref, wn2_ref, out_ref):
    acc = e_ref[:]
    for z_ref, w1_ref, b1_ref, w2_ref in (
            (zs_ref, ws1_ref, bs1_ref, ws2_ref),
            (za_ref, wa1_ref, ba1_ref, wa2_ref),
            (zn_ref, wn1_ref, bn1_ref, wn2_ref)):
        z = z_ref[0] + z_ref[1]
        h = _ssp(jnp.dot(z, w1_ref[:], preferred_element_type=jnp.float32) + b1_ref[:])
        acc = acc + jnp.dot(h, w2_ref[:], preferred_element_type=jnp.float32)
    out_ref[:] = acc


def _full(shape):
    return pl.BlockSpec(shape, lambda i: tuple(0 for _ in shape))


def _edge_mlp(dists2d, wts, e_total):
    grid = (e_total // EDGE_B,)
    in_specs = [pl.BlockSpec((EDGE_B, 1), lambda i: (i, 0)),
                _full((1, 128)), _full((1, 128)), _full((1, 128)),
                _full((128, 128)), _full((1, 128)),
                _full((128, 128)), _full((1, 128)),
                _full((128, 128))]
    return pl.pallas_call(
        _edge_body, grid=grid, in_specs=in_specs,
        out_specs=pl.BlockSpec((EDGE_B, 128), lambda i: (i, 0)),
        out_shape=jax.ShapeDtypeStruct((e_total, 128), jnp.float32),
    )(dists2d, jnp.asarray(_MUS), jnp.asarray(_ISIG2), jnp.asarray(_BMASK), *wts)


def _edge_mlp_ne(dists2d, senders2d, wts, y128, e_total):
    grid = (e_total // EDGE_B,)
    in_specs = [pl.BlockSpec((EDGE_B, 1), lambda i: (i, 0)),
                pl.BlockSpec((EDGE_B, 1), lambda i: (i, 0)),
                _full((1, 128)), _full((1, 128)), _full((1, 128)),
                _full((128, 128)), _full((1, 128)),
                _full((128, 128)), _full((1, 128)),
                _full((128, 128)), _full((128, 128))]
    return pl.pallas_call(
        _edge_ne_body, grid=grid, in_specs=in_specs,
        out_specs=pl.BlockSpec((EDGE_B, 128), lambda i: (i, 0)),
        out_shape=jax.ShapeDtypeStruct((e_total, 128), jnp.float32),
    )(dists2d, senders2d, jnp.asarray(_MUS), jnp.asarray(_ISIG2),
      jnp.asarray(_BMASK), *wts, y128)


def _node_h(elec, wts):
    grid = (N_ELEC // NODE_B,)
    in_specs = [pl.BlockSpec((NODE_B, 128), lambda i: (i, 0)),
                _full((128, 128)), _full((1, 128)), _full((128, 128)),
                _full((128, 128)), _full((1, 128)), _full((128, 128))]
    out_specs = [pl.BlockSpec((NODE_B, 128), lambda i: (i, 0))] * 2
    return pl.pallas_call(
        _node_h_body, grid=grid, in_specs=in_specs, out_specs=out_specs,
        out_shape=[jax.ShapeDtypeStruct((N_ELEC, 128), jnp.float32)] * 2,
    )(elec, *wts)


def _update(elec, zs, za, zn, wts):
    grid = (N_ELEC // NODE_B,)
    in_specs = [pl.BlockSpec((NODE_B, 128), lambda i: (i, 0))] + \
               [pl.BlockSpec((2, NODE_B, 128), lambda i: (0, i, 0))] * 3 + \
               [_full((128, 128)), _full((1, 128)), _full((128, 128))] * 3
    return pl.pallas_call(
        _update_body, grid=grid, in_specs=in_specs,
        out_specs=pl.BlockSpec((NODE_B, 128), lambda i: (i, 0)),
        out_shape=jax.ShapeDtypeStruct((N_ELEC, EMB), jnp.float32),
    )(elec, zs, za, zn, *wts)


def _sc_body(gather_flags):
  def outer_body(ws, wa, wn, h_s, h_a, ss3, sa3, rs3, ra3, rn3,
                 o_s, o_a, o_n, acc, ridx, sidx, bufw0, bufw1, bufh):
    @functools.partial(pl.run_scoped,
                       semw0=pltpu.SemaphoreType.DMA(()),
                       semw1=pltpu.SemaphoreType.DMA(()),
                       semh=pltpu.SemaphoreType.DMA(()))
    def body(semw0, semw1, semh):
      _sc_impl(gather_flags, ws, wa, wn, h_s, h_a, ss3, sa3, rs3, ra3, rn3,
               o_s, o_a, o_n, acc, ridx, sidx,
               bufw0, bufw1, bufh, semw0, semw1, semh)
  return outer_body


def _sc_impl(gather_flags, ws, wa, wn, h_s, h_a, ss3, sa3, rs3, ra3, rn3,
             o_s, o_a, o_n, acc, ridx, sidx,
             bufw0, bufw1, bufh, semw0, semw1, semh):
    c = lax.axis_index("c")
    s = lax.axis_index("s")
    w = s * 2 + c
    bufws = (bufw0, bufw1)
    semws = (semw0, semw1)

    def zero_buf_row(i, carry):
        for cc in range(8):
            bufw0[i, pl.ds(cc * 16, 16)] = jnp.zeros((16,), jnp.float32)
        return carry

    def _zero():
        for k in range(ZROWS // BATCH):
            pltpu.sync_copy(bufw0, acc.at[pl.ds(s * ZROWS + k * BATCH, BATCH)])

    def _writeout(o):
        pltpu.sync_copy(acc.at[pl.ds(s * ZROWS, ZROWS)],
                        o.at[c, pl.ds(s * ZROWS, ZROWS)])

    def _scatter(weh, r3, do_gather, h, s3):
        pltpu.sync_copy(r3.at[w], ridx)
        if do_gather:
            pltpu.sync_copy(s3.at[w], sidx)
        base0 = w * NBATCH

        def load_desc(j, k):
            return pltpu.make_async_copy(
                weh.at[pl.ds((base0 + j) * BATCH, BATCH)], bufws[k], semws[k])

        def gath_desc(j):
            return pltpu.make_async_copy(h.at[sidx.at[j]], bufh, semh)

        def do_batch(j, k):
            load_desc(j, k).wait()
            if do_gather:
                gath_desc(j).wait()

                def mul_row(i, carry2):
                    for cc in range(4):
                        sl = pl.ds(cc * 16, 16)
                        bufws[k][i, sl] = bufws[k][i, sl] * bufh[i, sl]
                    return carry2
                lax.fori_loop(0, BATCH, mul_row, 0)

                @pl.when(j + 1 < NBATCH)
                def _():
                    gath_desc(j + 1).start()
            pltpu.sync_copy(bufws[k], acc.at[ridx.at[j]], add=True)

            @pl.when(j + 2 < NBATCH)
            def _():
                load_desc(j + 2, k).start()

        load_desc(0, 0).start()
        load_desc(1, 1).start()
        if do_gather:
            gath_desc(0).start()

        def loop_body(tt, carry):
            do_batch(2 * tt, 0)
            do_batch(2 * tt + 1, 1)
            return carry
        lax.fori_loop(0, NBATCH // 2, loop_body, 0)

    for weh, r3, o, do_gather, h, s3 in (
            (ws, rs3, o_s, gather_flags[0], h_s, ss3),
            (wa, ra3, o_a, gather_flags[1], h_a, sa3),
            (wn, rn3, o_n, gather_flags[2], None, None)):
        lax.fori_loop(0, BATCH, zero_buf_row, 0)
        _zero()
        plsc.subcore_barrier()
        _scatter(weh, r3, do_gather, h, s3)
        plsc.subcore_barrier()
        _writeout(o)
        plsc.subcore_barrier()


def _make_sc(gather_flags, interpret=False):
  return functools.partial(
      pl.kernel,
      mesh=plsc.VectorSubcoreMesh(core_axis_name="c", subcore_axis_name="s"),
      out_type=[jax.ShapeDtypeStruct((2, NACC, 128), jnp.float32)] * 3,
      scratch_types=[pltpu.VMEM_SHARED((NACC, 128), jnp.float32),
                     pltpu.VMEM((NBATCH, BATCH), jnp.int32),
                     pltpu.VMEM((NBATCH, BATCH), jnp.int32),
                     pltpu.VMEM((BATCH, 128), jnp.float32),
                     pltpu.VMEM((BATCH, 128), jnp.float32),
                     pltpu.VMEM((BATCH, 128), jnp.float32)],
      interpret=interpret,
  )(_sc_body(gather_flags))


_seg3 = _make_sc((False, False, False))
_seg3g = _make_sc((True, True, False))


def _prep_w_mlp(layers, fold_row=None):
    w1 = _pad2(layers[0]['W'], 128, 128)
    b1 = _pad_row(layers[0]['b'], 128)
    w2 = _pad2(layers[1]['W'], 128, 128)
    b2 = _pad_row(layers[1]['b'], 128)
    w3 = layers[2]['W']
    if fold_row is not None:
        w3 = w3 * fold_row[None, :]
    w3 = _pad2(w3, 128, 128)
    return (w1, b1, w2, b2, w3)


def _prep_h_mlp(layers):
    return (_pad2(layers[0]['W'], 128, 128), _pad_row(layers[0]['b'], 128),
            _pad2(layers[1]['W'], 128, 128))


def _prep_g_mlp(layers):
    return (_pad2(layers[0]['W'], 128, 128), _pad_row(layers[0]['b'], 128),
            _pad2(layers[1]['W'], 128, 128))


def kernel(dists_same, dists_anti, dists_ne, senders_same, receivers_same,
           senders_anti, receivers_anti, senders_ne, receivers_ne, params):
    E = dists_same.shape[0]
    pad = EP - E

    def _pedge(x, val=0):
        return jnp.pad(x, (0, pad), constant_values=val)

    ds2 = _pedge(dists_same).reshape(EP, 1)
    da2 = _pedge(dists_anti).reshape(EP, 1)
    dn2 = _pedge(dists_ne).reshape(EP, 1)
    sn2 = _pedge(senders_ne).reshape(EP, 1)
    ss3 = _pedge(senders_same).reshape(NTILES, NBATCH, BATCH)
    sa3 = _pedge(senders_anti).reshape(NTILES, NBATCH, BATCH)
    rs3 = _pedge(receivers_same, N_ELEC).reshape(NTILES, NBATCH, BATCH)
    ra3 = _pedge(receivers_anti, N_ELEC).reshape(NTILES, NBATCH, BATCH)
    rn3 = _pedge(receivers_ne, N_ELEC).reshape(NTILES, NBATCH, BATCH)
    y128 = _pad2(params['Y'], 128, 128)

    elec = jnp.broadcast_to(params['X'][0], (N_ELEC, EMB))

    lps = params['layers']
    we = []
    for i in range(N_LAYERS):
        lp = lps[i]
        fold_s = lp['h_same'][0] if i == 0 else None
        fold_a = lp['h_anti'][0] if i == 0 else None
        we.append((_edge_mlp(ds2, _prep_w_mlp(lp['w_same'], fold_s), EP),
                   _edge_mlp(da2, _prep_w_mlp(lp['w_anti'], fold_a), EP),
                   _edge_mlp_ne(dn2, sn2, _prep_w_mlp(lp['w_ne']), y128, EP)))

    for i in range(N_LAYERS):
        lp = lps[i]
        we_s, we_a, weh_n = we[i]
        if i == 0:
            z_s, z_a, z_n = _seg3(we_s, we_a, weh_n, elec, elec,
                                  ss3, sa3, rs3, ra3, rn3)
        else:
            h_s, h_a = _node_h(elec, _prep_h_mlp(lp['h_same']) + _prep_h_mlp(lp['h_anti']))
            z_s, z_a, z_n = _seg3g(we_s, we_a, weh_n, h_s, h_a,
                                   ss3, sa3, rs3, ra3, rn3)
        elec = _update(elec, z_s, z_a, z_n,
                       _prep_g_mlp(lp['g_same']) + _prep_g_mlp(lp['g_anti'])
                       + _prep_g_mlp(lp['g_ne']))
    return elec


# speedup vs baseline: 1.4236x; 1.0094x over previous
import functools
import numpy as np
import jax
import jax.numpy as jnp
from jax import lax
from jax.experimental import pallas as pl
from jax.experimental.pallas import tpu as pltpu
from jax.experimental.pallas import tpu_sc as plsc

N_ELEC = 10000
N_NUC = 16
EMB = 128
KER = 64
DFD = 32
CUTOFF = 10.0
N_LAYERS = 3
LOG2 = float(np.log(2.0))

_delta = 1.0 / (2 * DFD)
_qs = np.linspace(_delta, 1.0 - _delta, DFD)
_mus = CUTOFF * _qs ** 2
_sigmas = (1.0 + CUTOFF * _qs) / 7.0
_MUS = np.zeros((1, 128), np.float32)
_MUS[0, :DFD] = _mus
_ISIG2 = np.zeros((1, 128), np.float32)
_ISIG2[0, :DFD] = 1.0 / _sigmas ** 2
_BMASK = np.zeros((1, 128), np.float32)
_BMASK[0, :DFD] = 1.0

EDGE_B = 2048
NODE_B = 2000

NTILES = 32
NBATCH = 80
BATCH = 64
EP = NTILES * NBATCH * BATCH
NACC = 10240
ZROWS = NACC // 16


def _ssp(x):
    return jnp.logaddexp(x, 0.0) - LOG2


def _pad2(w, r, c):
    return jnp.zeros((r, c), jnp.float32).at[: w.shape[0], : w.shape[1]].set(w)


def _pad_row(b, c):
    return jnp.zeros((1, c), jnp.float32).at[0, : b.shape[0]].set(b)


def _bdot(a, b_ref):
    return jnp.dot(a, b_ref[:], preferred_element_type=jnp.float32)


def _edge_body(d_ref, mus_ref, isig_ref, bmask_ref,
               w1_ref, b1_ref, w2_ref, b2_ref, w3_ref, out_ref):
    d = d_ref[:]
    env = d * d * jnp.exp(-d)
    t = d - mus_ref[:]
    feat = env * jnp.exp(-(t * t) * isig_ref[:]) * bmask_ref[:]
    h1 = _ssp(_bdot(feat, w1_ref) + b1_ref[:])
    h2 = _ssp(_bdot(h1, w2_ref) + b2_ref[:])
    out_ref[:] = _bdot(h2, w3_ref)


def _edge_ne_body(d_ref, s_ref, mus_ref, isig_ref, bmask_ref,
                  w1_ref, b1_ref, w2_ref, b2_ref, w3_ref, y_ref, out_ref):
    d = d_ref[:]
    env = d * d * jnp.exp(-d)
    t = d - mus_ref[:]
    feat = env * jnp.exp(-(t * t) * isig_ref[:]) * bmask_ref[:]
    h1 = _ssp(_bdot(feat, w1_ref) + b1_ref[:])
    h2 = _ssp(_bdot(h1, w2_ref) + b2_ref[:])
    we = _bdot(h2, w3_ref)
    s = s_ref[:]
    lanes = lax.broadcasted_iota(jnp.int32, (s.shape[0], 128), 1)
    onehot = (lanes == s).astype(jnp.float32)
    hx = jnp.dot(onehot, y_ref[:], preferred_element_type=jnp.float32)
    out_ref[:] = we * hx


def _node_h_body(e_ref, w1s_ref, b1s_ref, w2s_ref,
                 w1a_ref, b1a_ref, w2a_ref, hs_ref, ha_ref):
    e = e_ref[:]
    hs = _ssp(jnp.dot(e, w1s_ref[:], preferred_element_type=jnp.float32) + b1s_ref[:])
    hs_ref[:] = jnp.dot(hs, w2s_ref[:], preferred_element_type=jnp.float32)
    ha = _ssp(jnp.dot(e, w1a_ref[:], preferred_element_type=jnp.float32) + b1a_ref[:])
    ha_ref[:] = jnp.dot(ha, w2a_ref[:], preferred_element_type=jnp.float32)


def _update_body(e_ref, zs_ref, za_ref, zn_ref,
                 ws1_ref, bs1_ref, ws2_ref,
                 wa1_ref, ba1_ref, wa2_ref,
                 wn1_ref, bn1_ref, wn2_ref, out_ref):
    acc = e_ref[:]
    for z_ref, w1_ref, b1_ref, w2_ref in (
            (zs_ref, ws1_ref, bs1_ref, ws2_ref),
            (za_ref, wa1_ref, ba1_ref, wa2_ref),
            (zn_ref, wn1_ref, bn1_ref, wn2_ref)):
        z = z_ref[0] + z_ref[1]
        h = _ssp(jnp.dot(z, w1_ref[:], preferred_element_type=jnp.float32) + b1_ref[:])
        acc = acc + jnp.dot(h, w2_ref[:], preferred_element_type=jnp.float32)
    out_ref[:] = acc


def _full(shape):
    return pl.BlockSpec(shape, lambda i: tuple(0 for _ in shape))


def _edge_mlp(dists2d, wts, e_total):
    grid = (e_total // EDGE_B,)
    in_specs = [pl.BlockSpec((EDGE_B, 1), lambda i: (i, 0)),
                _full((1, 128)), _full((1, 128)), _full((1, 128)),
                _full((128, 128)), _full((1, 128)),
                _full((128, 128)), _full((1, 128)),
                _full((128, 128))]
    return pl.pallas_call(
        _edge_body, grid=grid, in_specs=in_specs,
        out_specs=pl.BlockSpec((EDGE_B, 128), lambda i: (i, 0)),
        out_shape=jax.ShapeDtypeStruct((e_total, 128), jnp.float32),
    )(dists2d, jnp.asarray(_MUS), jnp.asarray(_ISIG2), jnp.asarray(_BMASK), *wts)


def _edge_mlp_ne(dists2d, senders2d, wts, y128, e_total):
    grid = (e_total // EDGE_B,)
    in_specs = [pl.BlockSpec((EDGE_B, 1), lambda i: (i, 0)),
                pl.BlockSpec((EDGE_B, 1), lambda i: (i, 0)),
                _full((1, 128)), _full((1, 128)), _full((1, 128)),
                _full((128, 128)), _full((1, 128)),
                _full((128, 128)), _full((1, 128)),
                _full((128, 128)), _full((128, 128))]
    return pl.pallas_call(
        _edge_ne_body, grid=grid, in_specs=in_specs,
        out_specs=pl.BlockSpec((EDGE_B, 128), lambda i: (i, 0)),
        out_shape=jax.ShapeDtypeStruct((e_total, 128), jnp.float32),
    )(dists2d, senders2d, jnp.asarray(_MUS), jnp.asarray(_ISIG2),
      jnp.asarray(_BMASK), *wts, y128)


def _node_h(elec, wts):
    grid = (N_ELEC // NODE_B,)
    in_specs = [pl.BlockSpec((NODE_B, 128), lambda i: (i, 0)),
                _full((128, 128)), _full((1, 128)), _full((128, 128)),
                _full((128, 128)), _full((1, 128)), _full((128, 128))]
    out_specs = [pl.BlockSpec((NODE_B, 128), lambda i: (i, 0))] * 2
    return pl.pallas_call(
        _node_h_body, grid=grid, in_specs=in_specs, out_specs=out_specs,
        out_shape=[jax.ShapeDtypeStruct((N_ELEC, 128), jnp.float32)] * 2,
    )(elec, *wts)


def _update(elec, zs, za, zn, wts):
    grid = (N_ELEC // NODE_B,)
    in_specs = [pl.BlockSpec((NODE_B, 128), lambda i: (i, 0))] + \
               [pl.BlockSpec((2, NODE_B, 128), lambda i: (0, i, 0))] * 3 + \
               [_full((128, 128)), _full((1, 128)), _full((128, 128))] * 3
    return pl.pallas_call(
        _update_body, grid=grid, in_specs=in_specs,
        out_specs=pl.BlockSpec((NODE_B, 128), lambda i: (i, 0)),
        out_shape=jax.ShapeDtypeStruct((N_ELEC, EMB), jnp.float32),
    )(elec, zs, za, zn, *wts)


def _sc_body(gather_flags):
  def outer_body(ws, wa, wn, h_s, h_a, ss3, sa3, rs3, ra3, rn3,
                 o_s, o_a, o_n, acc, ridx, sidx, bufw0, bufw1, bufh):
    @functools.partial(pl.run_scoped,
                       semw0=pltpu.SemaphoreType.DMA(()),
                       semw1=pltpu.SemaphoreType.DMA(()),
                       semh=pltpu.SemaphoreType.DMA(()))
    def body(semw0, semw1, semh):
      _sc_impl(gather_flags, ws, wa, wn, h_s, h_a, ss3, sa3, rs3, ra3, rn3,
               o_s, o_a, o_n, acc, ridx, sidx,
               bufw0, bufw1, bufh, semw0, semw1, semh)
  return outer_body


def _sc_impl(gather_flags, ws, wa, wn, h_s, h_a, ss3, sa3, rs3, ra3, rn3,
             o_s, o_a, o_n, acc, ridx, sidx,
             bufw0, bufw1, bufh, semw0, semw1, semh):
    c = lax.axis_index("c")
    s = lax.axis_index("s")
    w = s * 2 + c
    bufws = (bufw0, bufw1)
    semws = (semw0, semw1)

    def zero_buf_row(i, carry):
        for cc in range(8):
            bufw0[i, pl.ds(cc * 16, 16)] = jnp.zeros((16,), jnp.float32)
        return carry

    def _zero():
        for k in range(ZROWS // BATCH):
            pltpu.sync_copy(bufw0, acc.at[pl.ds(s * ZROWS + k * BATCH, BATCH)])

    def _writeout(o):
        pltpu.sync_copy(acc.at[pl.ds(s * ZROWS, ZROWS)],
                        o.at[c, pl.ds(s * ZROWS, ZROWS)])

    def _scatter(weh, r3, do_gather, h, s3):
        pltpu.sync_copy(r3.at[w], ridx)
        if do_gather:
            pltpu.sync_copy(s3.at[w], sidx)
        base0 = w * NBATCH

        def load_desc(j, k):
            return pltpu.make_async_copy(
                weh.at[pl.ds((base0 + j) * BATCH, BATCH)], bufws[k], semws[k])

        def gath_desc(j):
            return pltpu.make_async_copy(h.at[sidx.at[j]], bufh, semh)

        def do_batch(j, k):
            load_desc(j, k).wait()
            if do_gather:
                gath_desc(j).wait()

                def mul_row(i, carry2):
                    for cc in range(4):
                        sl = pl.ds(cc * 16, 16)
                        bufws[k][i, sl] = bufws[k][i, sl] * bufh[i, sl]
                    return carry2
                lax.fori_loop(0, BATCH, mul_row, 0)

                @pl.when(j + 1 < NBATCH)
                def _():
                    gath_desc(j + 1).start()
            pltpu.sync_copy(bufws[k], acc.at[ridx.at[j]], add=True)

            @pl.when(j + 2 < NBATCH)
            def _():
                load_desc(j + 2, k).start()

        load_desc(0, 0).start()
        load_desc(1, 1).start()
        if do_gather:
            gath_desc(0).start()

        def loop_body(tt, carry):
            do_batch(2 * tt, 0)
            do_batch(2 * tt + 1, 1)
            return carry
        lax.fori_loop(0, NBATCH // 2, loop_body, 0)

    for weh, r3, o, do_gather, h, s3 in (
            (ws, rs3, o_s, gather_flags[0], h_s, ss3),
            (wa, ra3, o_a, gather_flags[1], h_a, sa3),
            (wn, rn3, o_n, gather_flags[2], None, None)):
        lax.fori_loop(0, BATCH, zero_buf_row, 0)
        _zero()
        plsc.subcore_barrier()
        _scatter(weh, r3, do_gather, h, s3)
        plsc.subcore_barrier()
        _writeout(o)
        plsc.subcore_barrier()


def _make_sc(gather_flags, interpret=False):
  return functools.partial(
      pl.kernel,
      mesh=plsc.VectorSubcoreMesh(core_axis_name="c", subcore_axis_name="s"),
      out_type=[jax.ShapeDtypeStruct((2, NACC, 128), jnp.float32)] * 3,
      scratch_types=[pltpu.VMEM_SHARED((NACC, 128), jnp.float32),
                     pltpu.VMEM((NBATCH, BATCH), jnp.int32),
                     pltpu.VMEM((NBATCH, BATCH), jnp.int32),
                     pltpu.VMEM((BATCH, 128), jnp.float32),
                     pltpu.VMEM((BATCH, 128), jnp.float32),
                     pltpu.VMEM((BATCH, 128), jnp.float32)],
      interpret=interpret,
  )(_sc_body(gather_flags))


_seg3 = _make_sc((False, False, False))
_seg3g = _make_sc((True, True, False))


def _prep_w_mlp(layers, fold_row=None):
    w1 = _pad2(layers[0]['W'], 128, 128)
    b1 = _pad_row(layers[0]['b'], 128)
    w2 = _pad2(layers[1]['W'], 128, 128)
    b2 = _pad_row(layers[1]['b'], 128)
    w3 = layers[2]['W']
    if fold_row is not None:
        w3 = w3 * fold_row[None, :]
    w3 = _pad2(w3, 128, 128)
    return (w1, b1, w2, b2, w3)


def _prep_h_mlp(layers):
    return (_pad2(layers[0]['W'], 128, 128), _pad_row(layers[0]['b'], 128),
            _pad2(layers[1]['W'], 128, 128))


def _prep_g_mlp(layers):
    return (_pad2(layers[0]['W'], 128, 128), _pad_row(layers[0]['b'], 128),
            _pad2(layers[1]['W'], 128, 128))


def kernel(dists_same, dists_anti, dists_ne, senders_same, receivers_same,
           senders_anti, receivers_anti, senders_ne, receivers_ne, params):
    E = dists_same.shape[0]
    pad = EP - E

    def _pedge(x, val=0):
        return jnp.pad(x, (0, pad), constant_values=val)

    ds2 = _pedge(dists_same).reshape(EP, 1)
    da2 = _pedge(dists_anti).reshape(EP, 1)
    dn2 = _pedge(dists_ne).reshape(EP, 1)
    sn2 = _pedge(senders_ne).reshape(EP, 1)
    ss3 = _pedge(senders_same).reshape(NTILES, NBATCH, BATCH)
    sa3 = _pedge(senders_anti).reshape(NTILES, NBATCH, BATCH)
    rs3 = _pedge(receivers_same, N_ELEC).reshape(NTILES, NBATCH, BATCH)
    ra3 = _pedge(receivers_anti, N_ELEC).reshape(NTILES, NBATCH, BATCH)
    rn3 = _pedge(receivers_ne, N_ELEC).reshape(NTILES, NBATCH, BATCH)
    y128 = _pad2(params['Y'], 128, 128)

    elec = jnp.broadcast_to(params['X'][0], (N_ELEC, EMB))

    lps = params['layers']
    we = []
    for i in range(N_LAYERS):
        lp = lps[i]
        fold_s = lp['h_same'][0] if i == 0 else None
        fold_a = lp['h_anti'][0] if i == 0 else None
        we.append((_edge_mlp(ds2, _prep_w_mlp(lp['w_same'], fold_s), EP),
                   _edge_mlp(da2, _prep_w_mlp(lp['w_anti'], fold_a), EP),
                   _edge_mlp_ne(dn2, sn2, _prep_w_mlp(lp['w_ne']), y128, EP)))

    for i in range(N_LAYERS):
        lp = lps[i]
        we_s, we_a, weh_n = we[i]
        if i == 0:
            z_s, z_a, z_n = _seg3(we_s, we_a, weh_n, elec, elec,
                                  ss3, sa3, rs3, ra3, rn3)
        else:
            h_s, h_a = _node_h(elec, _prep_h_mlp(lp['h_same']) + _prep_h_mlp(lp['h_anti']))
            z_s, z_a, z_n = _seg3g(we_s, we_a, weh_n, h_s, h_a,
                                   ss3, sa3, rs3, ra3, rn3)
        elec = _update(elec, z_s, z_a, z_n,
                       _prep_g_mlp(lp['g_same']) + _prep_g_mlp(lp['g_anti'])
                       + _prep_g_mlp(lp['g_ne']))
    return elec
